# Initial kernel scaffold; baseline (speedup 1.0000x reference)
import jax
import jax.numpy as jnp
from jax.experimental import pallas as pl


def kernel(graph_node_features, edge_index, edge_attr, W0, a0, W1, a1, W2, a2, We1, be1, We2, be2):
    raise NotImplementedError("write your pallas kernel here")


# speedup vs baseline: 5.7959x; 5.7959x over previous
import functools

import jax
import jax.numpy as jnp
from jax import lax
from jax.experimental import pallas as pl
from jax.experimental.pallas import tpu as pltpu
from jax.experimental.pallas import tpu_sc as plsc

N = 10000
E = 320000
DF = 128
DE = 16

NW = 32
CH = 64
NB = 10
CPB = 16
NCH = NB * CPB
EPT = NCH * CH
EPAD = NW * EPT
SENT = N
NPADM = 10016
AGGR = 10112
RPT = AGGR // 16

_LANE = 16


def _ew_body(ea_ref, w1t_ref, b1_ref, w2_ref, b2_ref, out_ref):
    h = jnp.maximum(ea_ref[...] @ w1t_ref[...] + b1_ref[...], 0.0)
    out_ref[...] = jnp.sum(h * w2_ref[...], axis=1, keepdims=True) + b2_ref[...]


def _edge_weights(edge_attr, We1, be1, We2, be2):
    blk = 3200
    grid = E // blk
    return pl.pallas_call(
        _ew_body,
        grid=(grid,),
        in_specs=[
            pl.BlockSpec((blk, DE), lambda i: (i, 0)),
            pl.BlockSpec((DE, DF), lambda i: (0, 0)),
            pl.BlockSpec((1, DF), lambda i: (0, 0)),
            pl.BlockSpec((1, DF), lambda i: (0, 0)),
            pl.BlockSpec((1, 1), lambda i: (0, 0)),
        ],
        out_specs=pl.BlockSpec((blk, 1), lambda i: (i, 0)),
        out_shape=jax.ShapeDtypeStruct((E, 1), jnp.float32),
    )(edge_attr, We1.T, be1.reshape(1, DF), We2, be2.reshape(1, 1))


def _mm_body(h_ref, wt_ref, a2_ref, wh_ref, s2_ref):
    wh = h_ref[...] @ wt_ref[...]
    wh_ref[...] = wh
    s2_ref[...] = wh @ a2_ref[...]


def _project(h, W, a2pad):
    blk = 1000
    grid = N // blk
    return pl.pallas_call(
        _mm_body,
        grid=(grid,),
        in_specs=[
            pl.BlockSpec((blk, DF), lambda i: (i, 0)),
            pl.BlockSpec((DF, DF), lambda i: (0, 0)),
            pl.BlockSpec((DF, DF), lambda i: (0, 0)),
        ],
        out_specs=[
            pl.BlockSpec((blk, DF), lambda i: (i, 0)),
            pl.BlockSpec((blk, DF), lambda i: (i, 0)),
        ],
        out_shape=[
            jax.ShapeDtypeStruct((N, DF), jnp.float32),
            jax.ShapeDtypeStruct((N, DF), jnp.float32),
        ],
    )(h, W.T, a2pad)


def _comb_body(mT_ref, agg_ref, out_ref, *, apply_elu):
    m = jnp.max(mT_ref[...], axis=1, keepdims=True)
    M = jnp.max(m)
    e = jnp.exp(m - M)
    sm = e / jnp.sum(e)
    agg = agg_ref[0, :N, :] + agg_ref[1, :N, :]
    out = sm * agg
    if apply_elu:
        out = jnp.where(out > 0, out, jnp.exp(out) - 1.0)
    out_ref[...] = out


def _combine(mT, agg, apply_elu):
    return pl.pallas_call(
        functools.partial(_comb_body, apply_elu=apply_elu),
        out_shape=jax.ShapeDtypeStruct((N, DF), jnp.float32),
    )(mT, agg)


_sc_mesh = plsc.VectorSubcoreMesh(core_axis_name="c", subcore_axis_name="s")


@functools.partial(
    pl.kernel,
    out_type=(
        jax.ShapeDtypeStruct((NW, NPADM), jnp.float32),
        jax.ShapeDtypeStruct((2, AGGR, DF), jnp.float32),
    ),
    mesh=_sc_mesh,
    compiler_params=pltpu.CompilerParams(needs_layout_passes=False),
    scratch_types=[
        pltpu.VMEM((NPADM,), jnp.float32),
        pltpu.VMEM((NPADM,), jnp.float32),
        pltpu.VMEM((NPADM,), jnp.float32),
        pltpu.VMEM((CPB, CH), jnp.int32),
        pltpu.VMEM((CPB, CH), jnp.int32),
        pltpu.VMEM((CPB, CH), jnp.float32),
        pltpu.VMEM((CH, DF), jnp.float32),
        pltpu.VMEM_SHARED((AGGR, DF), jnp.float32),
        pltpu.SemaphoreType.DMA,
    ],
)
def _sc_edges(wh_hbm, ssrc_hbm, sdst_hbm, src_hbm, dst_hbm, ew_hbm,
              zer_hbm, ninf_hbm, m_out, agg_out,
              ssrc_v, sdst_v, m_v, src_v, dst_v, ew_v, rows_v, agg_sh, sem):
    c = lax.axis_index("c")
    s = lax.axis_index("s")
    w = s * 2 + c
    pltpu.sync_copy(ssrc_hbm, ssrc_v)
    pltpu.sync_copy(sdst_hbm, sdst_v)
    pltpu.sync_copy(ninf_hbm, m_v)
    pltpu.sync_copy(zer_hbm, agg_sh.at[pl.ds(s * RPT, RPT)])
    plsc.subcore_barrier()

    def chunk(i, carry):
        cp = pltpu.async_copy(wh_hbm.at[src_v.at[i]], rows_v, sem)
        for g in range(CH // _LANE):
            off = g * _LANE
            src16 = src_v.at[i][pl.ds(off, _LANE)]
            dst16 = dst_v.at[i][pl.ds(off, _LANE)]
            ew16 = ew_v.at[i][pl.ds(off, _LANE)]
            t = (plsc.load_gather(ssrc_v, [src16])
                 + plsc.load_gather(sdst_v, [dst16]))
            sc = jnp.where(t >= 0, t, 0.2 * t) * ew16
            cnt, _ = plsc.scan_count(dst16)
            mx = jnp.max(cnt)

            def occ(j, c2, cnt=cnt, dst16=dst16, sc=sc):
                mj = cnt == j + 1
                cur = plsc.load_gather(m_v, [dst16], mask=mj)
                plsc.store_scatter(m_v, [dst16], jnp.maximum(cur, sc),
                                   mask=mj)
                return c2

            lax.fori_loop(0, mx, occ, 0)
        cp.wait()
        pltpu.sync_copy(rows_v, agg_sh.at[dst_v.at[i]], add=True)
        return carry

    for b in range(NB):
        row0 = w * NCH + b * CPB
        pltpu.sync_copy(src_hbm.at[pl.ds(row0, CPB)], src_v)
        pltpu.sync_copy(dst_hbm.at[pl.ds(row0, CPB)], dst_v)
        pltpu.sync_copy(ew_hbm.at[pl.ds(row0, CPB)], ew_v)
        lax.fori_loop(0, CPB, chunk, 0)
    plsc.subcore_barrier()
    pltpu.sync_copy(m_v, m_out.at[w])
    pltpu.sync_copy(agg_sh.at[pl.ds(s * RPT, RPT)],
                    agg_out.at[c].at[pl.ds(s * RPT, RPT)])


def kernel(graph_node_features, edge_index, edge_attr,
           W0, a0, W1, a1, W2, a2, We1, be1, We2, be2):
    src = edge_index[0]
    dst = edge_index[1]
    npad = EPAD - E
    src_p = jnp.concatenate([src, jnp.zeros((npad,), jnp.int32)]
                            ).reshape(EPAD // CH, CH)
    dst_p = jnp.concatenate([dst, jnp.full((npad,), SENT, jnp.int32)]
                            ).reshape(EPAD // CH, CH)

    ew = _edge_weights(edge_attr, We1, be1, We2, be2)
    ew_p = jnp.concatenate([ew[:, 0], jnp.zeros((npad,), jnp.float32)]
                           ).reshape(EPAD // CH, CH)

    zer = jnp.zeros((RPT, DF), jnp.float32)
    ninf = jnp.full((NPADM,), -jnp.inf, jnp.float32)

    def a2pad_of(a):
        a2 = jnp.zeros((DF, DF), jnp.float32)
        a2 = a2.at[:, 0].set(a[0, 0, :DF])
        return a2.at[:, 1].set(a[0, 0, DF:])

    h = graph_node_features
    for li, (W, a) in enumerate(((W0, a0), (W1, a1), (W2, a2))):
        wh, s2 = _project(h, W, a2pad_of(a))
        ssrc = jnp.pad(s2[:, 0], (0, NPADM - N))
        sdst = jnp.pad(s2[:, 1], (0, NPADM - N))
        m_part, agg = _sc_edges(wh, ssrc, sdst, src_p, dst_p, ew_p, zer, ninf)
        mT = m_part.T[:N]
        h = _combine(mT, agg, apply_elu=(li < 2))
    return h
